# Initial kernel scaffold; baseline (speedup 1.0000x reference)
import jax
import jax.numpy as jnp
from jax.experimental import pallas as pl


def kernel(x, edge_index, edge_attr, batch, gammas, W1a, b1a, W1b, b1b, W2a, b2a, W2b, b2b, W3a, b3a, W3b, b3b, W4a, b4a, W4b, b4b):
    raise NotImplementedError("write your pallas kernel here")


# speedup vs baseline: 1.5928x; 1.5928x over previous
import functools

import jax
import jax.numpy as jnp
from jax.experimental import pallas as pl
from jax.experimental.pallas import tpu as pltpu

_NEG = -1e30


def _gat_block(xi_ref, xj_ref, ea_ref, gec_ref, ger_ref,
               w1a_ref, b1a_ref, w1b_ref, b1b_ref,
               w2a_ref, b2a_ref, w2b_ref, b2b_ref,
               w3a_ref, b3a_ref, w3b_ref, b3b_ref,
               w4a_ref, b4a_ref, w4b_ref, b4b_ref, gt_ref,
               out_ref, m_acc, s_acc, *, nb, W):
    i = pl.program_id(0)

    @pl.when(i == 0)
    def _init():
        m_acc[...] = jnp.full_like(m_acc, _NEG)
        s_acc[...] = jnp.zeros_like(s_acc)

    f32 = jnp.float32
    dot = functools.partial(jax.lax.dot_general, preferred_element_type=f32)

    z = jnp.concatenate([xi_ref[...], xj_ref[...], ea_ref[...]], axis=1)
    h1 = jnp.maximum(dot(z, w1a_ref[...], (((1,), (0,)), ((), ()))) + b1a_ref[...], 0.0)
    q = dot(h1, w1b_ref[...], (((1,), (0,)), ((), ()))) + b1b_ref[...]
    r2 = jnp.maximum(dot(q, w2a_ref[...], (((1,), (0,)), ((), ()))) + b2a_ref[...], 0.0)
    raw = dot(r2, w2b_ref[...], (((1,), (0,)), ((), ()))) + b2b_ref[...]
    m3 = jnp.maximum(dot(q, w3a_ref[...], (((1,), (0,)), ((), ()))) + b3a_ref[...], 0.0)
    msg = dot(m3, w3b_ref[...], (((1,), (0,)), ((), ()))) + b3b_ref[...]

    B = z.shape[0]
    ge_col = gec_ref[...]
    ge_row = ger_ref[0]
    oh = (ge_col == jax.lax.broadcasted_iota(jnp.int32, (B, W), 1).astype(f32))
    ohT_f = (jax.lax.broadcasted_iota(jnp.int32, (W, B), 0).astype(f32) == ge_row).astype(f32)
    oh_f = oh.astype(f32)

    mb = jnp.max(jnp.where(oh, raw, _NEG), axis=0, keepdims=True)
    newm = jnp.maximum(m_acc[...], mb)
    scale = jnp.exp(m_acc[...] - newm)
    m_g = dot(oh_f, newm, (((1,), (1,)), ((), ())))
    w = jnp.exp(raw - m_g)
    vals = jnp.concatenate([w, w * msg], axis=1)
    contrib = dot(vals, ohT_f, (((0,), (1,)), ((), ())))
    m_acc[...] = newm
    s_acc[...] = s_acc[...] * scale + contrib

    @pl.when(i == nb - 1)
    def _final():
        s = s_acc[...]
        d = s[0:1, :]
        ok = d > 0.0
        node_T = jnp.where(ok, s[1:17, :] / jnp.where(ok, d, 1.0), 0.0)
        combined = jnp.concatenate([node_T, gt_ref[...]], axis=0)
        hT = jnp.maximum(dot(w4a_ref[...], combined, (((1,), (0,)), ((), ()))) + b4a_ref[...], 0.0)
        outT = dot(w4b_ref[...], hT, (((1,), (0,)), ((), ()))) + b4b_ref[...]
        out_ref[...] = outT


def kernel(x, edge_index, edge_attr, batch, gammas,
           W1a, b1a, W1b, b1b, W2a, b2a, W2b, b2b,
           W3a, b3a, W3b, b3b, W4a, b4a, W4b, b4b):
    f32 = jnp.float32
    N = x.shape[0]
    E = edge_index.shape[1]
    G = gammas.shape[0]
    W = max(128, ((G + 1 + 127) // 128) * 128)

    src = edge_index[0]
    dst = edge_index[1]
    is_first = jnp.concatenate([jnp.ones((1,), jnp.bool_), jnp.diff(batch) != 0])
    seg_node = jnp.where(is_first, batch, G).astype(f32)
    seg_e = seg_node[src]

    xi = x[src]
    xj = x[dst]

    B = 2000
    while E % B != 0:
        B //= 2
    nb = E // B

    ge_col = seg_e[:, None]
    ge_row = seg_e.reshape(nb, 1, B)
    gt = jnp.zeros((2, W), f32).at[:, :G].set(gammas.T)

    row = lambda b: b.reshape(1, -1)
    col = lambda b: b.reshape(-1, 1)

    full = lambda *shape: pl.BlockSpec(shape, lambda i: (0,) * len(shape))
    outT = pl.pallas_call(
        functools.partial(_gat_block, nb=nb, W=W),
        grid=(nb,),
        in_specs=[
            pl.BlockSpec((B, 3), lambda i: (i, 0)),
            pl.BlockSpec((B, 3), lambda i: (i, 0)),
            pl.BlockSpec((B, 5), lambda i: (i, 0)),
            pl.BlockSpec((B, 1), lambda i: (i, 0)),
            pl.BlockSpec((1, 1, B), lambda i: (i, 0, 0)),
            full(11, 64), full(1, 64), full(64, 16), full(1, 16),
            full(16, 16), full(1, 16), full(16, 1), full(1, 1),
            full(16, 64), full(1, 64), full(64, 16), full(1, 16),
            full(64, 18), full(64, 1), full(2, 64), full(2, 1),
            full(2, W),
        ],
        out_specs=pl.BlockSpec((2, W), lambda i: (0, 0)),
        out_shape=jax.ShapeDtypeStruct((2, W), f32),
        scratch_shapes=[pltpu.VMEM((1, W), f32), pltpu.VMEM((17, W), f32)],
    )(xi, xj, edge_attr, ge_col, ge_row,
      W1a.T, row(b1a), W1b.T, row(b1b),
      W2a.T, row(b2a), W2b.T, row(b2b),
      W3a.T, row(b3a), W3b.T, row(b3b),
      W4a, col(b4a), W4b, col(b4b), gt)

    return outT[:, :G].T


# speedup vs baseline: 6.8628x; 4.3086x over previous
import functools

import jax
import jax.numpy as jnp
from jax import lax
from jax.experimental import pallas as pl
from jax.experimental.pallas import tpu as pltpu
from jax.experimental.pallas import tpu_sc as plsc

_NEG = -1e30

_NC, _NS, _L = 2, 16, 16
_NW = _NC * _NS

_N, _E, _G = 100000, 1600000, 1000
_EPW = _E // _NW
_CH = 2000
_NCHUNK = _EPW // _CH
_CBUF = 2048
_REG = 50176
_CTOT = _NW * _REG
_NF = 11

_BT = 1024
_W = 1024


def _sc_body(src_h, dst_h, x0, x1, x2, e0, e1, e2, e3, e4, tbl_h,
             *rest):
    outs = rest[:12]
    ocnt = rest[12]
    tbl_v, src_v, dst_v, csrc, cdst, ce, cg = rest[13:20]
    fv = rest[20:20 + _NF]
    cnt_v = rest[20 + _NF]
    sem = rest[21 + _NF]

    i32 = jnp.int32
    f32 = jnp.float32
    wid = lax.axis_index("s") * _NC + lax.axis_index("c")
    wbase = wid * _REG

    pltpu.sync_copy(tbl_h, tbl_v)

    def z(i, _):
        zv = jnp.zeros((_L,), i32)
        csrc[pl.ds(i * _L, _L)] = zv
        cdst[pl.ds(i * _L, _L)] = zv
        ce[pl.ds(i * _L, _L)] = zv
        return 0
    lax.fori_loop(0, _CBUF // _L, z, 0)

    lanes = lax.iota(i32, _L)

    def chunk(c, woff):
        base_e = pl.multiple_of(wid * _EPW + c * _CH, 16)
        pltpu.sync_copy(src_h.at[pl.ds(base_e, _CH)], src_v)
        pltpu.sync_copy(dst_h.at[pl.ds(base_e, _CH)], dst_v)

        def vec(i, cnt):
            s16 = src_v[pl.ds(i * _L, _L)]
            d16 = dst_v[pl.ds(i * _L, _L)]
            w32 = plsc.load_gather(tbl_v, [lax.shift_right_logical(s16, 1)])
            hi = lax.shift_right_logical(w32, 16)
            seg16 = jnp.where((s16 & 1) == 0, w32 & 0xFFFF, hi & 0xFFFF)
            m = seg16 > 0
            plsc.store_compressed(csrc.at[pl.ds(cnt, _L)], s16, mask=m)
            plsc.store_compressed(cdst.at[pl.ds(cnt, _L)], d16, mask=m)
            plsc.store_compressed(ce.at[pl.ds(cnt, _L)], base_e + i * _L + lanes, mask=m)
            plsc.store_compressed(cg.at[pl.ds(cnt, _L)], (seg16 - 1).astype(f32), mask=m)
            return cnt + jnp.sum(m.astype(i32))

        cnt = lax.fori_loop(0, _CH // _L, vec, jnp.int32(0))

        pad = (8 - (cnt & 7)) & 7
        mpad = lanes < pad
        plsc.store_compressed(csrc.at[pl.ds(cnt, _L)], jnp.zeros((_L,), i32), mask=mpad)
        plsc.store_compressed(cdst.at[pl.ds(cnt, _L)], jnp.zeros((_L,), i32), mask=mpad)
        plsc.store_compressed(ce.at[pl.ds(cnt, _L)], jnp.zeros((_L,), i32), mask=mpad)
        plsc.store_compressed(cg.at[pl.ds(cnt, _L)], jnp.full((_L,), float(_G), f32), mask=mpad)
        cnt_p = cnt + pad

        def gw(j, _):
            sl = pl.ds(pl.multiple_of(j * 128, 128), 128)
            cps = []
            tables = [x0, x1, x2, x0, x1, x2, e0, e1, e2, e3, e4]
            idxs = [csrc, csrc, csrc, cdst, cdst, cdst, ce, ce, ce, ce, ce]
            for f in range(_NF):
                cps.append(pltpu.async_copy(tables[f].at[idxs[f].at[sl]],
                                            fv[f].at[sl], sem))
            for cp in cps:
                cp.wait()
            wo = pl.multiple_of(wbase + woff + j * 128, 8)
            for f in range(_NF):
                pltpu.sync_copy(fv[f].at[sl], outs[f].at[pl.ds(wo, 128)])
            pltpu.sync_copy(cg.at[sl], outs[_NF].at[pl.ds(wo, 128)])
            return 0

        lax.fori_loop(0, (cnt_p + 127) // 128, gw, 0)
        return woff + cnt_p

    total = lax.fori_loop(0, _NCHUNK, chunk, jnp.int32(0))
    cnt_v[...] = jnp.where(lanes == 0, total, 0)
    pltpu.sync_copy(cnt_v, ocnt.at[pl.ds(pl.multiple_of(wid * _L, 16), _L)])


def _sc_compact(src, dst, xf, ef, tblp):
    f32 = jnp.float32
    i32 = jnp.int32
    mesh = plsc.VectorSubcoreMesh(core_axis_name="c", subcore_axis_name="s",
                                  num_cores=_NC, num_subcores=_NS)
    return pl.kernel(
        _sc_body,
        out_type=[jax.ShapeDtypeStruct((_CTOT,), f32) for _ in range(12)]
                 + [jax.ShapeDtypeStruct((_NW * _L,), i32)],
        mesh=mesh,
        compiler_params=pltpu.CompilerParams(needs_layout_passes=False),
        scratch_types=[
            pltpu.VMEM((_N // 2,), i32),
            pltpu.VMEM((_CH,), i32),
            pltpu.VMEM((_CH,), i32),
            pltpu.VMEM((_CBUF,), i32),
            pltpu.VMEM((_CBUF,), i32),
            pltpu.VMEM((_CBUF,), i32),
            pltpu.VMEM((_CBUF,), f32),
        ] + [pltpu.VMEM((_CBUF,), f32) for _ in range(_NF)] + [
            pltpu.VMEM((_L,), i32),
            pltpu.SemaphoreType.DMA,
        ],
    )(src, dst, *xf, *ef, tblp)


def _tc_body(*refs):
    fld = refs[:12]
    ocnt = refs[12]
    (w1a_ref, b1a_ref, w1b_ref, b1b_ref,
     w2a_ref, b2a_ref, w2b_ref, b2b_ref,
     w3a_ref, b3a_ref, w3b_ref, b3b_ref,
     w4a_ref, b4a_ref, w4b_ref, b4b_ref, gt_ref) = refs[13:30]
    out_ref = refs[30]
    fv = refs[31:43]
    m_acc, s_acc = refs[43], refs[44]
    sem = refs[45]

    f32 = jnp.float32
    i32 = jnp.int32
    r = pl.program_id(0)
    dot = functools.partial(lax.dot_general, preferred_element_type=f32)

    @pl.when(r == 0)
    def _init():
        m_acc[...] = jnp.full_like(m_acc, _NEG)
        s_acc[...] = jnp.zeros_like(s_acc)

    cnt_r = ocnt[r, 0]

    def blk(k, _):
        off = pl.multiple_of(r * _REG + k * _BT, 8)
        cps = [pltpu.make_async_copy(fld[f].at[pl.ds(off, _BT), :], fv[f], sem)
               for f in range(12)]
        for cp in cps:
            cp.start()
        for cp in cps:
            cp.wait()

        nval = cnt_r - k * _BT
        rowid = lax.broadcasted_iota(i32, (_BT, 1), 0)
        valid = rowid < nval

        z = jnp.concatenate([fv[f][...] for f in range(_NF)], axis=1)
        h1 = jnp.maximum(dot(z, w1a_ref[...], (((1,), (0,)), ((), ()))) + b1a_ref[...], 0.0)
        q = dot(h1, w1b_ref[...], (((1,), (0,)), ((), ()))) + b1b_ref[...]
        r2 = jnp.maximum(dot(q, w2a_ref[...], (((1,), (0,)), ((), ()))) + b2a_ref[...], 0.0)
        raw = dot(r2, w2b_ref[...], (((1,), (0,)), ((), ()))) + b2b_ref[...]
        m3 = jnp.maximum(dot(q, w3a_ref[...], (((1,), (0,)), ((), ()))) + b3a_ref[...], 0.0)
        msg = dot(m3, w3b_ref[...], (((1,), (0,)), ((), ()))) + b3b_ref[...]

        ge_col = fv[_NF][...]
        oh = (ge_col == lax.broadcasted_iota(i32, (_BT, _W), 1).astype(f32)) & valid
        oh_f = oh.astype(f32)
        mb = jnp.max(jnp.where(oh, raw, _NEG), axis=0, keepdims=True)
        newm = jnp.maximum(m_acc[...], mb)
        scale = jnp.exp(m_acc[...] - newm)
        m_g = dot(oh_f, newm, (((1,), (1,)), ((), ())))
        wgt = jnp.where(valid, jnp.exp(raw - m_g), 0.0)
        msg_m = jnp.where(valid, msg, 0.0)
        vals = jnp.concatenate([wgt, wgt * msg_m], axis=1)
        contrib = dot(vals, oh_f, (((0,), (0,)), ((), ())))
        m_acc[...] = newm
        s_acc[...] = s_acc[...] * scale + contrib
        return 0

    lax.fori_loop(0, (cnt_r + _BT - 1) // _BT, blk, 0)

    @pl.when(r == _NW - 1)
    def _final():
        s = s_acc[...]
        d = s[0:1, :]
        ok = d > 0.0
        node_T = jnp.where(ok, s[1:17, :] / jnp.where(ok, d, 1.0), 0.0)
        combined = jnp.concatenate([node_T, gt_ref[...]], axis=0)
        hT = jnp.maximum(dot(w4a_ref[...], combined, (((1,), (0,)), ((), ()))) + b4a_ref[...], 0.0)
        out_ref[...] = dot(w4b_ref[...], hT, (((1,), (0,)), ((), ()))) + b4b_ref[...]


def kernel(x, edge_index, edge_attr, batch, gammas,
           W1a, b1a, W1b, b1b, W2a, b2a, W2b, b2b,
           W3a, b3a, W3b, b3b, W4a, b4a, W4b, b4b):
    f32 = jnp.float32
    i32 = jnp.int32

    src = edge_index[0]
    dst = edge_index[1]
    is_first = jnp.concatenate([jnp.ones((1,), jnp.bool_), jnp.diff(batch) != 0])
    tbl = jnp.where(is_first, batch + 1, 0).astype(i32)
    tblp = tbl[0::2] | (tbl[1::2] << 16)

    xf = [x[:, k] + 0.0 for k in range(3)]
    ef = [edge_attr[:, k] + 0.0 for k in range(5)]

    outs = _sc_compact(src, dst, xf, ef, tblp)
    fields, ocnt = outs[:12], outs[12]

    gt = jnp.zeros((2, _W), f32).at[:, :_G].set(gammas.T)
    row = lambda b: b.reshape(1, -1)
    col = lambda b: b.reshape(-1, 1)
    vspec = pl.BlockSpec(memory_space=pltpu.MemorySpace.VMEM)
    aspec = pl.BlockSpec(memory_space=pl.ANY)

    outT = pl.pallas_call(
        _tc_body,
        grid=(_NW,),
        in_specs=[aspec] * 12
                 + [pl.BlockSpec(memory_space=pltpu.MemorySpace.SMEM)]
                 + [vspec] * 17,
        out_specs=pl.BlockSpec(memory_space=pltpu.MemorySpace.VMEM),
        out_shape=jax.ShapeDtypeStruct((2, _W), f32),
        scratch_shapes=[pltpu.VMEM((_BT, 1), f32) for _ in range(12)] + [
            pltpu.VMEM((1, _W), f32), pltpu.VMEM((17, _W), f32),
            pltpu.SemaphoreType.DMA,
        ],
    )(*[f.reshape(_CTOT, 1) for f in fields], ocnt.reshape(_NW, _L),
      W1a.T, row(b1a), W1b.T, row(b1b),
      W2a.T, row(b2a), W2b.T, row(b2b),
      W3a.T, row(b3a), W3b.T, row(b3b),
      W4a, col(b4a), W4b, col(b4b), gt)

    return outT[:, :_G].T


# speedup vs baseline: 13.1152x; 1.9111x over previous
import functools

import jax
import jax.numpy as jnp
from jax import lax
from jax.experimental import pallas as pl
from jax.experimental.pallas import tpu as pltpu
from jax.experimental.pallas import tpu_sc as plsc

_NEG = -1e30

_NC, _NS, _L = 2, 16, 16
_NW = _NC * _NS

_N, _E, _G = 100000, 1600000, 1000
_EPW = _E // _NW
_CH = 2000
_NCHUNK = _EPW // _CH
_CBUF = 2048
_REG = 50176
_CTOT = _NW * _REG
_NF = 11

_BT = 1024
_W = 1024


def _sc_body(src_h, dst_h, x0, x1, x2, ea_h, tbl_h,
             *rest):
    outs = rest[:12]
    ocnt = rest[12]
    tbl_v, src_v, dst_v, csrc, cdst, cg = rest[13:19]
    ce5 = rest[19:24]
    fv = rest[24:24 + _NF]
    cnt_v = rest[24 + _NF]
    sem = rest[25 + _NF]
    semw = rest[26 + _NF]

    i32 = jnp.int32
    f32 = jnp.float32
    wid = lax.axis_index("s") * _NC + lax.axis_index("c")
    wbase = wid * _REG

    pltpu.sync_copy(tbl_h, tbl_v)

    def z(i, _):
        zv = jnp.zeros((_L,), i32)
        csrc[pl.ds(i * _L, _L)] = zv
        cdst[pl.ds(i * _L, _L)] = zv
        for b in ce5:
            b[pl.ds(i * _L, _L)] = zv
        return 0
    lax.fori_loop(0, _CBUF // _L, z, 0)

    lanes = lax.iota(i32, _L)

    def chunk(c, woff):
        base_e = pl.multiple_of(wid * _EPW + c * _CH, 16)
        s0 = pltpu.async_copy(src_h.at[pl.ds(base_e, _CH)], src_v, sem)
        s1 = pltpu.async_copy(dst_h.at[pl.ds(base_e, _CH)], dst_v, sem)
        s0.wait()
        s1.wait()

        def vec(i, cnt):
            s16 = src_v[pl.ds(i * _L, _L)]
            d16 = dst_v[pl.ds(i * _L, _L)]
            w32 = plsc.load_gather(tbl_v, [lax.shift_right_logical(s16, 1)])
            hi = lax.shift_right_logical(w32, 16)
            seg16 = jnp.where((s16 & 1) == 0, w32 & 0xFFFF, hi & 0xFFFF)
            m = seg16 > 0
            plsc.store_compressed(csrc.at[pl.ds(cnt, _L)], s16, mask=m)
            plsc.store_compressed(cdst.at[pl.ds(cnt, _L)], d16, mask=m)
            e5 = (base_e + i * _L + lanes) * 5
            for k in range(5):
                plsc.store_compressed(ce5[k].at[pl.ds(cnt, _L)], e5 + k, mask=m)
            plsc.store_compressed(cg.at[pl.ds(cnt, _L)], (seg16 - 1).astype(f32), mask=m)
            return cnt + jnp.sum(m.astype(i32))

        cnt = lax.fori_loop(0, _CH // _L, vec, jnp.int32(0))

        pad = (8 - (cnt & 7)) & 7
        mpad = lanes < pad
        plsc.store_compressed(csrc.at[pl.ds(cnt, _L)], jnp.zeros((_L,), i32), mask=mpad)
        plsc.store_compressed(cdst.at[pl.ds(cnt, _L)], jnp.zeros((_L,), i32), mask=mpad)
        for k in range(5):
            plsc.store_compressed(ce5[k].at[pl.ds(cnt, _L)], jnp.zeros((_L,), i32), mask=mpad)
        plsc.store_compressed(cg.at[pl.ds(cnt, _L)], jnp.full((_L,), float(_G), f32), mask=mpad)
        cnt_p = cnt + pad

        def gw(j, _):
            sl = pl.ds(pl.multiple_of(j * 128, 128), 128)
            cps = []
            tables = [x0, x1, x2, x0, x1, x2, ea_h, ea_h, ea_h, ea_h, ea_h]
            idxs = [csrc, csrc, csrc, cdst, cdst, cdst] + list(ce5)
            for f in range(_NF):
                cps.append(pltpu.async_copy(tables[f].at[idxs[f].at[sl]],
                                            fv[f].at[sl], sem))
            for cp in cps:
                cp.wait()
            wo = pl.multiple_of(wbase + woff + j * 128, 8)
            wps = [pltpu.async_copy(fv[f].at[sl], outs[f].at[pl.ds(wo, 128)], semw)
                   for f in range(_NF)]
            wps.append(pltpu.async_copy(cg.at[sl], outs[_NF].at[pl.ds(wo, 128)], semw))
            for wp in wps:
                wp.wait()
            return 0

        lax.fori_loop(0, (cnt_p + 127) // 128, gw, 0)
        return woff + cnt_p

    total = lax.fori_loop(0, _NCHUNK, chunk, jnp.int32(0))
    cnt_v[...] = jnp.where(lanes == 0, total, 0)
    pltpu.sync_copy(cnt_v, ocnt.at[pl.ds(pl.multiple_of(wid * _L, 16), _L)])


def _sc_compact(src, dst, xf, ea_flat, tblp):
    f32 = jnp.float32
    i32 = jnp.int32
    mesh = plsc.VectorSubcoreMesh(core_axis_name="c", subcore_axis_name="s",
                                  num_cores=_NC, num_subcores=_NS)
    return pl.kernel(
        _sc_body,
        out_type=[jax.ShapeDtypeStruct((_CTOT,), f32) for _ in range(12)]
                 + [jax.ShapeDtypeStruct((_NW * _L,), i32)],
        mesh=mesh,
        compiler_params=pltpu.CompilerParams(needs_layout_passes=False),
        scratch_types=[
            pltpu.VMEM((_N // 2,), i32),
            pltpu.VMEM((_CH,), i32),
            pltpu.VMEM((_CH,), i32),
            pltpu.VMEM((_CBUF,), i32),
            pltpu.VMEM((_CBUF,), i32),
            pltpu.VMEM((_CBUF,), f32),
        ] + [pltpu.VMEM((_CBUF,), i32) for _ in range(5)]
          + [pltpu.VMEM((_CBUF,), f32) for _ in range(_NF)] + [
            pltpu.VMEM((_L,), i32),
            pltpu.SemaphoreType.DMA,
            pltpu.SemaphoreType.DMA,
        ],
    )(src, dst, *xf, ea_flat, tblp)


def _tc_body(*refs):
    fld = refs[:12]
    ocnt = refs[12]
    (w1a_ref, b1a_ref, w1b_ref, b1b_ref,
     w2a_ref, b2a_ref, w2b_ref, b2b_ref,
     w3a_ref, b3a_ref, w3b_ref, b3b_ref,
     w4at_ref, b4a_ref, w4bt_ref, b4b_ref, gam_ref) = refs[13:30]
    out_ref = refs[30]
    fv = refs[31:43]
    m_acc, s_acc = refs[43], refs[44]
    sem = refs[45]

    f32 = jnp.float32
    i32 = jnp.int32
    r = pl.program_id(0)
    dot = functools.partial(lax.dot_general, preferred_element_type=f32)

    @pl.when(r == 0)
    def _init():
        m_acc[...] = jnp.full_like(m_acc, _NEG)
        s_acc[...] = jnp.zeros_like(s_acc)

    cnt_r = ocnt[r, 0]

    def blk(k, _):
        off = pl.multiple_of(r * _REG + k * _BT, 8)
        cps = [pltpu.make_async_copy(fld[f].at[pl.ds(off, _BT)], fv[f].at[0], sem)
               for f in range(12)]
        for cp in cps:
            cp.start()
        for cp in cps:
            cp.wait()

        nval = cnt_r - k * _BT
        valid = lax.broadcasted_iota(i32, (1, _BT), 1) < nval

        zT = jnp.concatenate([fv[f][...] for f in range(_NF)], axis=0)
        h1 = jnp.maximum(dot(w1a_ref[...], zT, (((1,), (0,)), ((), ()))) + b1a_ref[...], 0.0)
        q = dot(w1b_ref[...], h1, (((1,), (0,)), ((), ()))) + b1b_ref[...]
        r2 = jnp.maximum(dot(w2a_ref[...], q, (((1,), (0,)), ((), ()))) + b2a_ref[...], 0.0)
        raw = dot(w2b_ref[...], r2, (((1,), (0,)), ((), ()))) + b2b_ref[...]
        m3 = jnp.maximum(dot(w3a_ref[...], q, (((1,), (0,)), ((), ()))) + b3a_ref[...], 0.0)
        msg = dot(w3b_ref[...], m3, (((1,), (0,)), ((), ()))) + b3b_ref[...]

        ge_row = fv[_NF][...]
        oh = (lax.broadcasted_iota(i32, (_W, _BT), 0).astype(f32) == ge_row) & valid
        oh_f = oh.astype(f32)
        mb = jnp.max(jnp.where(oh, raw, _NEG), axis=1, keepdims=True)
        newm = jnp.maximum(m_acc[...], mb)
        scale = jnp.exp(m_acc[...] - newm)
        m_g = dot(newm, oh_f, (((0,), (0,)), ((), ())))
        wgt = jnp.where(valid, jnp.exp(raw - m_g), 0.0)
        msg_m = jnp.where(valid, msg, 0.0)
        vals = jnp.concatenate([wgt, wgt * msg_m], axis=0)
        contrib = dot(oh_f, vals, (((1,), (1,)), ((), ())))
        m_acc[...] = newm
        s_acc[...] = s_acc[...] * scale + contrib
        return 0

    lax.fori_loop(0, (cnt_r + _BT - 1) // _BT, blk, 0)

    @pl.when(r == _NW - 1)
    def _final():
        s = s_acc[...]
        d = s[:, 0:1]
        ok = d > 0.0
        node = jnp.where(ok, s[:, 1:17] / jnp.where(ok, d, 1.0), 0.0)
        combined = jnp.concatenate([node[:_G, :], gam_ref[...]], axis=1)
        h4 = jnp.maximum(dot(combined, w4at_ref[...], (((1,), (0,)), ((), ()))) + b4a_ref[...], 0.0)
        out_ref[...] = dot(h4, w4bt_ref[...], (((1,), (0,)), ((), ()))) + b4b_ref[...]


def kernel(x, edge_index, edge_attr, batch, gammas,
           W1a, b1a, W1b, b1b, W2a, b2a, W2b, b2b,
           W3a, b3a, W3b, b3b, W4a, b4a, W4b, b4b):
    f32 = jnp.float32
    i32 = jnp.int32

    src = edge_index[0]
    dst = edge_index[1]
    is_first = jnp.concatenate([jnp.ones((1,), jnp.bool_), jnp.diff(batch) != 0])
    tbl = jnp.where(is_first, batch + 1, 0).astype(i32)
    tblp = tbl[0::2] | (tbl[1::2] << 16)

    xf = [x[:, k] + 0.0 for k in range(3)]
    ea_flat = edge_attr.reshape(-1)

    outs = _sc_compact(src, dst, xf, ea_flat, tblp)
    fields, ocnt = outs[:12], outs[12]

    col = lambda b: b.reshape(-1, 1)
    row = lambda b: b.reshape(1, -1)
    vspec = pl.BlockSpec(memory_space=pltpu.MemorySpace.VMEM)
    aspec = pl.BlockSpec(memory_space=pl.ANY)

    out = pl.pallas_call(
        _tc_body,
        grid=(_NW,),
        in_specs=[aspec] * 12
                 + [pl.BlockSpec(memory_space=pltpu.MemorySpace.SMEM)]
                 + [vspec] * 17,
        out_specs=pl.BlockSpec(memory_space=pltpu.MemorySpace.VMEM),
        out_shape=jax.ShapeDtypeStruct((_G, 2), f32),
        scratch_shapes=[pltpu.VMEM((1, _BT), f32) for _ in range(12)] + [
            pltpu.VMEM((_W, 1), f32), pltpu.VMEM((_W, 17), f32),
            pltpu.SemaphoreType.DMA,
        ],
    )(*fields, ocnt.reshape(_NW, _L),
      W1a, col(b1a), W1b, col(b1b),
      W2a, col(b2a), W2b, col(b2b),
      W3a, col(b3a), W3b, col(b3b),
      W4a.T, row(b4a), W4b.T, row(b4b), gammas)

    return out
